# baseline (speedup 1.0000x reference)
import functools

import jax
import jax.numpy as jnp
from jax import lax
from jax.experimental import pallas as pl
from jax.experimental.pallas import tpu as pltpu
from jax.experimental.pallas import tpu_sc as plsc

_CHUNK = 128


@functools.cache
def _make_gather(num_workers, chunks, d):
    mesh = plsc.VectorSubcoreMesh(core_axis_name="c", subcore_axis_name="s")
    info = plsc.get_sparse_core_info()
    nc = info.num_cores
    per_w = chunks * _CHUNK
    pack = _CHUNK // d
    groups = _CHUNK // 16
    shift = pack.bit_length() - 1

    @functools.partial(
        pl.kernel,
        mesh=mesh,
        out_type=jax.ShapeDtypeStruct((d, num_workers * per_w), jnp.float32),
        scratch_types=[
            pltpu.VMEM((chunks, _CHUNK), jnp.int32),
            pltpu.VMEM((chunks, _CHUNK), jnp.int32),
            pltpu.VMEM((chunks, _CHUNK, _CHUNK), jnp.float32),
            pltpu.VMEM((d, per_w), jnp.float32),
            pltpu.SemaphoreType.DMA,
        ],
        compiler_params=pltpu.CompilerParams(
            use_tc_tiling_on_sc=True, needs_layout_passes=False
        ),
    )
    def gather(ids_hbm, table_hbm, out_hbm, idx_v, q_v, buf_v, emb_v, sem):
        wid = lax.axis_index("s") * nc + lax.axis_index("c")
        base = wid * per_w
        for j in range(chunks):
            pltpu.sync_copy(ids_hbm.at[wid, pl.ds(j * _CHUNK, _CHUNK)], idx_v.at[j])
        for j in range(chunks):
            for g in range(groups):
                sl = pl.ds(g * 16, 16)
                vv = idx_v[j, sl]
                q_v[j, sl] = (vv & (pack - 1)) * d
                idx_v[j, sl] = vv >> shift
        copies = [
            pltpu.async_copy(table_hbm.at[idx_v.at[j]], buf_v.at[j], sem)
            for j in range(chunks)
        ]
        for c in copies:
            c.wait()

        rows = [lax.iota(jnp.int32, 16) + g * 16 for g in range(groups)]

        def step(c, _):
            for j in range(chunks):
                for g in range(groups):
                    sl = pl.ds(g * 16, 16)
                    cols = q_v[j, sl] + c
                    val = plsc.load_gather(buf_v.at[j], [rows[g], cols])
                    emb_v[c, pl.ds(j * _CHUNK + g * 16, 16)] = val
            return _

        lax.fori_loop(0, d, step, 0)
        pltpu.sync_copy(emb_v, out_hbm.at[:, pl.ds(base, per_w)])

    return gather


@functools.cache
def _make_transpose(v, d, w):
    mesh = plsc.VectorSubcoreMesh(core_axis_name="c", subcore_axis_name="s")
    info = plsc.get_sparse_core_info()
    nc = info.num_cores
    nw = nc * info.num_subcores
    pack = _CHUNK // d
    r_per = w // pack
    ns_full = v // w
    tail_w = v - ns_full * w
    tail_r = tail_w // pack
    ns = ns_full + (1 if tail_w else 0)
    niter = (ns + nw - 1) // nw

    @functools.partial(
        pl.kernel,
        mesh=mesh,
        out_type=jax.ShapeDtypeStruct((v // pack, _CHUNK), jnp.float32),
        scratch_types=[
            pltpu.VMEM((2, d, w), jnp.float32),
            pltpu.VMEM((2, r_per, _CHUNK), jnp.float32),
            pltpu.VMEM((max(tail_w, 1), d), jnp.float32),
            pltpu.SemaphoreType.DMA,
            pltpu.SemaphoreType.DMA,
        ],
        compiler_params=pltpu.CompilerParams(
            use_tc_tiling_on_sc=True, needs_layout_passes=False
        ),
    )
    def transp(tab_hbm, tail_hbm, out_hbm, tbuf, obuf, sbuf, rsem, wsem):
        wid = lax.axis_index("s") * nc + lax.axis_index("c")
        e_half = [lax.iota(jnp.int32, 16) + half * 16 for half in range(d // 16)]

        def fire_read(s2, buf):
            @pl.when(s2 < ns_full)
            def _():
                pltpu.async_copy(tab_hbm.at[:, pl.ds(s2 * w, w)], tbuf.at[buf], rsem)

        def compute(cur, i, s, rows, width):
            @pl.when(i >= 2)
            def _():
                pltpu.make_async_copy(
                    obuf.at[cur], out_hbm.at[pl.ds(0, r_per), :], wsem
                ).wait()

            for r in range(rows):
                for g in range(_CHUNK // 16):
                    u = jnp.full((16,), r * pack + (g >> 1), jnp.int32)
                    val = plsc.load_gather(tbuf.at[cur], [e_half[g & 1], u])
                    obuf[cur, r, pl.ds(g * 16, 16)] = val
            pltpu.async_copy(
                obuf.at[cur, pl.ds(0, rows), :],
                out_hbm.at[pl.ds(s * r_per, rows), :],
                wsem,
            )

        fire_read(wid, 0)

        def body(i, carry):
            s = wid + nw * i
            cur = i & 1

            @pl.when(s < ns_full)
            def _():
                pltpu.make_async_copy(
                    tab_hbm.at[:, pl.ds(0, w)], tbuf.at[cur], rsem
                ).wait()
                fire_read(s + nw, 1 - cur)
                compute(cur, i, s, r_per, w)

            if tail_w:
                @pl.when(s == ns_full)
                def _():
                    pltpu.sync_copy(tail_hbm, sbuf)

                    @pl.when(i >= 2)
                    def _():
                        pltpu.make_async_copy(
                            obuf.at[cur], out_hbm.at[pl.ds(0, r_per), :], wsem
                        ).wait()

                    for r in range(tail_r):
                        for g in range(_CHUNK // 16):
                            u = jnp.full((16,), r * pack + (g >> 1), jnp.int32)
                            val = plsc.load_gather(sbuf, [u, e_half[g & 1]])
                            obuf[cur, r, pl.ds(g * 16, 16)] = val
                    pltpu.async_copy(
                        obuf.at[cur, pl.ds(0, tail_r), :],
                        out_hbm.at[pl.ds(ns_full * r_per, tail_r), :],
                        wsem,
                    )

            return carry

        lax.fori_loop(0, niter, body, 0)
        last_s = wid + nw * (niter - 1)
        pltpu.make_async_copy(
            obuf.at[0], out_hbm.at[pl.ds(0, r_per), :], wsem
        ).wait()

        @pl.when(last_s != ns_full)
        def _():
            pltpu.make_async_copy(
                obuf.at[0], out_hbm.at[pl.ds(0, r_per), :], wsem
            ).wait()

        if tail_w:
            @pl.when(last_s == ns_full)
            def _():
                pltpu.make_async_copy(
                    obuf.at[0, pl.ds(0, tail_r), :],
                    out_hbm.at[pl.ds(0, tail_r), :],
                    wsem,
                ).wait()

    return transp


def _x_body(d, f, x_ref, o_ref):
    o_ref[0, pl.ds(d, f), :] = jnp.swapaxes(x_ref[0], 0, 1)


def _emb_body(o1_ref, e_ref, o_ref):
    o_ref[0] = e_ref[...]


def kernel(ids, x, table):
    b, h, f = x.shape
    v, d = table.shape
    ids32 = ids.astype(jnp.int32)

    info = plsc.get_sparse_core_info()
    nw = info.num_cores * info.num_subcores
    per_w = b // nw
    chunks = per_w // _CHUNK
    pack = _CHUNK // d
    ids2 = ids32.reshape(nw, per_w)
    tail_u = v % _CHUNK
    tail64 = table[v - tail_u :, :]
    table_r = _make_transpose(v, d, _CHUNK)(table.T, tail64)
    emb_t = _make_gather(nw, chunks, d)(ids2, table_r)

    rows = d + f
    x_t = x.transpose(1, 0, 2)

    bl = 4096
    out_x = pl.pallas_call(
        functools.partial(_x_body, d, f),
        grid=(h, b // bl),
        in_specs=[pl.BlockSpec((1, bl, f), lambda i, j: (i, j, 0))],
        out_specs=pl.BlockSpec((1, rows, bl), lambda i, j: (i, 0, j)),
        out_shape=jax.ShapeDtypeStruct((h, rows, b), jnp.float32),
        compiler_params=pltpu.CompilerParams(
            dimension_semantics=("arbitrary", "arbitrary"),
        ),
    )(x_t)

    bl2 = 4096
    out3 = pl.pallas_call(
        _emb_body,
        grid=(b // bl2, h),
        in_specs=[
            pl.BlockSpec(memory_space=pltpu.MemorySpace.HBM),
            pl.BlockSpec((d, bl2), lambda j, i: (0, j)),
        ],
        out_specs=pl.BlockSpec((1, d, bl2), lambda j, i: (i, 0, j)),
        out_shape=jax.ShapeDtypeStruct((h, rows, b), jnp.float32),
        input_output_aliases={0: 0},
        compiler_params=pltpu.CompilerParams(
            dimension_semantics=("arbitrary", "arbitrary"),
        ),
    )(out_x, emb_t)

    return out3.transpose(2, 0, 1)


# speedup vs baseline: 1.4036x; 1.4036x over previous
import functools

import jax
import jax.numpy as jnp
from jax import lax
from jax.experimental import pallas as pl
from jax.experimental.pallas import tpu as pltpu
from jax.experimental.pallas import tpu_sc as plsc

_CHUNK = 128


@functools.cache
def _make_gather(num_workers, chunks, d):
    mesh = plsc.VectorSubcoreMesh(core_axis_name="c", subcore_axis_name="s")
    info = plsc.get_sparse_core_info()
    nc = info.num_cores
    per_w = chunks * _CHUNK
    pack = _CHUNK // d
    groups = _CHUNK // 16
    shift = pack.bit_length() - 1

    @functools.partial(
        pl.kernel,
        mesh=mesh,
        out_type=jax.ShapeDtypeStruct((d, num_workers * per_w), jnp.float32),
        scratch_types=[
            pltpu.VMEM((chunks, _CHUNK), jnp.int32),
            pltpu.VMEM((chunks, _CHUNK), jnp.int32),
            pltpu.VMEM((chunks, _CHUNK, _CHUNK), jnp.float32),
            pltpu.VMEM((d, per_w), jnp.float32),
            pltpu.SemaphoreType.DMA,
        ],
        compiler_params=pltpu.CompilerParams(
            use_tc_tiling_on_sc=True, needs_layout_passes=False
        ),
    )
    def gather(ids_hbm, table_hbm, out_hbm, idx_v, q_v, buf_v, emb_v, sem):
        wid = lax.axis_index("s") * nc + lax.axis_index("c")
        base = wid * per_w
        for j in range(chunks):
            pltpu.sync_copy(ids_hbm.at[wid, pl.ds(j * _CHUNK, _CHUNK)], idx_v.at[j])
        for j in range(chunks):
            for g in range(groups):
                sl = pl.ds(g * 16, 16)
                vv = idx_v[j, sl]
                q_v[j, sl] = (vv & (pack - 1)) * d
                idx_v[j, sl] = vv >> shift
        copies = [
            pltpu.async_copy(table_hbm.at[idx_v.at[j]], buf_v.at[j], sem)
            for j in range(chunks)
        ]
        for c in copies:
            c.wait()

        rows = [lax.iota(jnp.int32, 16) + g * 16 for g in range(groups)]

        def step(c, _):
            for j in range(chunks):
                for g in range(groups):
                    sl = pl.ds(g * 16, 16)
                    cols = q_v[j, sl] + c
                    val = plsc.load_gather(buf_v.at[j], [rows[g], cols])
                    emb_v[c, pl.ds(j * _CHUNK + g * 16, 16)] = val
            return _

        lax.fori_loop(0, d, step, 0)
        pltpu.sync_copy(emb_v, out_hbm.at[:, pl.ds(base, per_w)])

    return gather


def _x_body(d, f, x_ref, o_ref):
    o_ref[0, pl.ds(d, f), :] = jnp.swapaxes(x_ref[0], 0, 1)


def _emb_body(o1_ref, e_ref, o_ref):
    o_ref[0] = e_ref[...]


def kernel(ids, x, table):
    b, h, f = x.shape
    v, d = table.shape
    ids32 = ids.astype(jnp.int32)

    info = plsc.get_sparse_core_info()
    nw = info.num_cores * info.num_subcores
    per_w = b // nw
    chunks = per_w // _CHUNK
    pack = _CHUNK // d
    ids2 = ids32.reshape(nw, per_w)
    table_r = table.reshape(v // pack, _CHUNK)
    emb_t = _make_gather(nw, chunks, d)(ids2, table_r)

    rows = d + f
    x_t = x.transpose(1, 0, 2)

    bl = 4096
    out_x = pl.pallas_call(
        functools.partial(_x_body, d, f),
        grid=(h, b // bl),
        in_specs=[pl.BlockSpec((1, bl, f), lambda i, j: (i, j, 0))],
        out_specs=pl.BlockSpec((1, rows, bl), lambda i, j: (i, 0, j)),
        out_shape=jax.ShapeDtypeStruct((h, rows, b), jnp.float32),
        compiler_params=pltpu.CompilerParams(
            dimension_semantics=("arbitrary", "arbitrary"),
        ),
    )(x_t)

    bl2 = 4096
    out3 = pl.pallas_call(
        _emb_body,
        grid=(b // bl2, h),
        in_specs=[
            pl.BlockSpec(memory_space=pltpu.MemorySpace.HBM),
            pl.BlockSpec((d, bl2), lambda j, i: (0, j)),
        ],
        out_specs=pl.BlockSpec((1, d, bl2), lambda j, i: (i, 0, j)),
        out_shape=jax.ShapeDtypeStruct((h, rows, b), jnp.float32),
        input_output_aliases={0: 0},
        compiler_params=pltpu.CompilerParams(
            dimension_semantics=("arbitrary", "arbitrary"),
        ),
    )(out_x, emb_t)

    return out3.transpose(2, 0, 1)


# speedup vs baseline: 1.5906x; 1.1333x over previous
import functools

import jax
import jax.numpy as jnp
from jax import lax
from jax.experimental import pallas as pl
from jax.experimental.pallas import tpu as pltpu
from jax.experimental.pallas import tpu_sc as plsc

_CHUNK = 128


@functools.cache
def _make_gather(num_workers, chunks, d):
    mesh = plsc.VectorSubcoreMesh(core_axis_name="c", subcore_axis_name="s")
    info = plsc.get_sparse_core_info()
    nc = info.num_cores
    per_w = chunks * _CHUNK
    pack = _CHUNK // d
    groups = _CHUNK // 16
    shift = pack.bit_length() - 1

    @functools.partial(
        pl.kernel,
        mesh=mesh,
        out_type=jax.ShapeDtypeStruct((d, num_workers * per_w), jnp.float32),
        scratch_types=[
            pltpu.VMEM((chunks, _CHUNK), jnp.int32),
            pltpu.VMEM((chunks, _CHUNK), jnp.int32),
            pltpu.VMEM((chunks, _CHUNK, _CHUNK), jnp.float32),
            pltpu.VMEM((d, per_w), jnp.float32),
            pltpu.SemaphoreType.DMA,
        ],
        compiler_params=pltpu.CompilerParams(
            use_tc_tiling_on_sc=True, needs_layout_passes=False
        ),
    )
    def gather(ids_hbm, table_hbm, out_hbm, idx_v, q_v, buf_v, emb_v, sem):
        wid = lax.axis_index("s") * nc + lax.axis_index("c")
        base = wid * per_w
        for j in range(chunks):
            pltpu.sync_copy(ids_hbm.at[wid, pl.ds(j * _CHUNK, _CHUNK)], idx_v.at[j])
        for j in range(chunks):
            for g in range(groups):
                sl = pl.ds(g * 16, 16)
                vv = idx_v[j, sl]
                q_v[j, sl] = (vv & (pack - 1)) * d
                idx_v[j, sl] = vv >> shift
        copies = [
            pltpu.async_copy(table_hbm.at[idx_v.at[j]], buf_v.at[j], sem)
            for j in range(chunks)
        ]
        for c in copies:
            c.wait()

        rows = [lax.iota(jnp.int32, 16) + g * 16 for g in range(groups)]

        def step(c, _):
            for j in range(chunks):
                for g in range(groups):
                    sl = pl.ds(g * 16, 16)
                    cols = q_v[j, sl] + c
                    val = plsc.load_gather(buf_v.at[j], [rows[g], cols])
                    emb_v[c, pl.ds(j * _CHUNK + g * 16, 16)] = val
            return _

        lax.fori_loop(0, d, step, 0)
        pltpu.sync_copy(emb_v, out_hbm.at[:, pl.ds(base, per_w)])

    return gather


@functools.cache
def _make_depad(v, d, k_tiles):
    mesh = plsc.VectorSubcoreMesh(core_axis_name="c", subcore_axis_name="s")
    info = plsc.get_sparse_core_info()
    nc = info.num_cores
    nw = nc * info.num_subcores
    pack = _CHUNK // d
    nt = v // 8
    r_per = k_tiles * 8 * d // _CHUNK
    nslab = nt // k_tiles
    niter = (nslab + nw - 1) // nw

    @functools.partial(
        pl.kernel,
        mesh=mesh,
        out_type=jax.ShapeDtypeStruct((v // pack, _CHUNK), jnp.float32),
        scratch_types=[
            pltpu.VMEM((2, k_tiles, 8, d), jnp.float32),
            pltpu.VMEM((2, r_per, _CHUNK), jnp.float32),
            pltpu.SemaphoreType.DMA,
            pltpu.SemaphoreType.DMA,
        ],
        compiler_params=pltpu.CompilerParams(
            use_tc_tiling_on_sc=True, needs_layout_passes=False
        ),
    )
    def depad(tab_hbm, out_hbm, vbuf, obuf, rsem, wsem):
        wid = lax.axis_index("s") * nc + lax.axis_index("c")

        def fire_read(s2, buf):
            @pl.when(s2 < nslab)
            def _():
                pltpu.async_copy(
                    tab_hbm.at[pl.ds(s2 * k_tiles, k_tiles)], vbuf.at[buf], rsem
                )

        fire_read(wid, 0)

        def body(i, carry):
            s = wid + nw * i
            cur = i & 1

            @pl.when(s < nslab)
            def _():
                pltpu.make_async_copy(
                    tab_hbm.at[pl.ds(0, k_tiles)], vbuf.at[cur], rsem
                ).wait()
                fire_read(s + nw, 1 - cur)

                @pl.when(i >= 2)
                def _():
                    pltpu.make_async_copy(
                        obuf.at[cur], out_hbm.at[pl.ds(0, r_per), :], wsem
                    ).wait()

                for r in range(r_per):
                    for m in range(_CHUNK // 16):
                        half = m & (d // 16 - 1)
                        q = m // (d // 16)
                        obuf[cur, r, pl.ds(m * 16, 16)] = vbuf[
                            cur,
                            r // 2,
                            (r & 1) * pack + q,
                            pl.ds(half * 16, 16),
                        ]
                pltpu.async_copy(
                    obuf.at[cur], out_hbm.at[pl.ds(s * r_per, r_per), :], wsem
                )

            return carry

        lax.fori_loop(0, niter, body, 0)
        for _tail in range(2):
            pltpu.make_async_copy(
                obuf.at[0], out_hbm.at[pl.ds(0, r_per), :], wsem
            ).wait()

    return depad


def _x_body(d, f, x_ref, o_ref):
    o_ref[0, pl.ds(d, f), :] = jnp.swapaxes(x_ref[0], 0, 1)


def _emb_body(o1_ref, e_ref, o_ref):
    o_ref[0] = e_ref[...]


def kernel(ids, x, table):
    b, h, f = x.shape
    v, d = table.shape
    ids32 = ids.astype(jnp.int32)

    info = plsc.get_sparse_core_info()
    nw = info.num_cores * info.num_subcores
    per_w = b // nw
    chunks = per_w // _CHUNK
    pack = _CHUNK // d
    ids2 = ids32.reshape(nw, per_w)
    table3 = table.reshape(v // 8, 8, d)
    table_r = _make_depad(v, d, 40)(table3)
    emb_t = _make_gather(nw, chunks, d)(ids2, table_r)

    rows = d + f
    x_t = x.transpose(1, 0, 2)

    bl = 4096
    out_x = pl.pallas_call(
        functools.partial(_x_body, d, f),
        grid=(h, b // bl),
        in_specs=[pl.BlockSpec((1, bl, f), lambda i, j: (i, j, 0))],
        out_specs=pl.BlockSpec((1, rows, bl), lambda i, j: (i, 0, j)),
        out_shape=jax.ShapeDtypeStruct((h, rows, b), jnp.float32),
        compiler_params=pltpu.CompilerParams(
            dimension_semantics=("arbitrary", "arbitrary"),
        ),
    )(x_t)

    bl2 = 4096
    out3 = pl.pallas_call(
        _emb_body,
        grid=(b // bl2, h),
        in_specs=[
            pl.BlockSpec(memory_space=pltpu.MemorySpace.HBM),
            pl.BlockSpec((d, bl2), lambda j, i: (0, j)),
        ],
        out_specs=pl.BlockSpec((1, d, bl2), lambda j, i: (i, 0, j)),
        out_shape=jax.ShapeDtypeStruct((h, rows, b), jnp.float32),
        input_output_aliases={0: 0},
        compiler_params=pltpu.CompilerParams(
            dimension_semantics=("arbitrary", "arbitrary"),
        ),
    )(out_x, emb_t)

    return out3.transpose(2, 0, 1)
